# baseline (speedup 1.0000x reference)
import functools

import jax
import jax.numpy as jnp
from jax import lax
from jax.experimental import pallas as pl
from jax.experimental.pallas import tpu as pltpu
from jax.experimental.pallas import tpu_sc as plsc

_B = 16384
_D = 32
_NC = 2
_NS = 16
_NW = _NC * _NS
_BPW = _B // _NW
_CHUNK = 128
_NCH = _BPW // _CHUNK
_NG = _BPW // 16


def _embed(sid, cid, semb, cemb):
    mesh = plsc.VectorSubcoreMesh(core_axis_name="c", subcore_axis_name="s")

    @functools.partial(
        pl.kernel,
        mesh=mesh,
        out_type=jax.ShapeDtypeStruct((_B, 2 * _D), jnp.float32),
        compiler_params=pltpu.CompilerParams(
            use_tc_tiling_on_sc=False, needs_layout_passes=False),
        scratch_types=[
            pltpu.VMEM((_BPW,), jnp.int32),
            pltpu.VMEM((_BPW,), jnp.int32),
            pltpu.VMEM((8, _D), jnp.float32),
            pltpu.VMEM((_BPW, _D), jnp.float32),
            pltpu.VMEM((_BPW, _D), jnp.float32),
            pltpu.SemaphoreType.DMA,
        ],
    )
    def body(sid_hbm, cid_hbm, semb_hbm, cemb_hbm, out_hbm,
             sidx, cidx, sv, srows, crows, sem):
        wid = lax.axis_index("s") * _NC + lax.axis_index("c")
        base = wid * _BPW
        pltpu.sync_copy(cid_hbm.at[wid], cidx)
        copies = []
        for j in range(_NCH):
            copies.append(pltpu.async_copy(
                cemb_hbm.at[cidx.at[pl.ds(j * _CHUNK, _CHUNK)]],
                crows.at[pl.ds(j * _CHUNK, _CHUNK)], sem))
        pltpu.sync_copy(sid_hbm.at[wid], sidx)
        pltpu.sync_copy(semb_hbm, sv)

        cols = [jnp.full((16,), c, jnp.int32) for c in range(_D)]

        def group(g, _):
            sid16 = sidx[pl.ds(g * 16, 16)]
            rows16 = lax.iota(jnp.int32, 16) + g * 16
            for c in range(_D):
                vals = plsc.load_gather(sv, [sid16, cols[c]])
                plsc.store_scatter(srows, [rows16, cols[c]], vals)
            return _

        lax.fori_loop(0, _NG, group, None)
        pltpu.sync_copy(srows, out_hbm.at[pl.ds(base, _BPW), pl.ds(0, _D)])
        for c in copies:
            c.wait()
        pltpu.sync_copy(crows, out_hbm.at[pl.ds(base, _BPW), pl.ds(_D, _D)])

    return body(sid, cid, semb, cemb)


def kernel(strategy_id, cause_index, strategy_emb, cause_emb):
    sid = strategy_id.astype(jnp.int32).reshape(_NW, _BPW)
    cid = cause_index.astype(jnp.int32).reshape(_NW, _BPW)
    return _embed(sid, cid, strategy_emb, cause_emb)


# speedup vs baseline: 1.0298x; 1.0298x over previous
import functools

import jax
import jax.numpy as jnp
from jax import lax
from jax.experimental import pallas as pl
from jax.experimental.pallas import tpu as pltpu
from jax.experimental.pallas import tpu_sc as plsc

_B = 16384
_D = 32
_NC = 2
_NS = 16
_NW = _NC * _NS
_BPW = _B // _NW
_CHUNK = 128
_NCH = _BPW // _CHUNK
_NG = _BPW // 16


def _embed(sid, cid, semb, cemb):
    mesh = plsc.VectorSubcoreMesh(core_axis_name="c", subcore_axis_name="s")

    @functools.partial(
        pl.kernel,
        mesh=mesh,
        out_type=jax.ShapeDtypeStruct((2 * _D, _B), jnp.float32),
        compiler_params=pltpu.CompilerParams(
            use_tc_tiling_on_sc=False, needs_layout_passes=False),
        scratch_types=[
            pltpu.VMEM((_BPW,), jnp.int32),
            pltpu.VMEM((_BPW,), jnp.int32),
            pltpu.VMEM((8, _D), jnp.float32),
            pltpu.VMEM((_BPW, _D), jnp.float32),
            pltpu.VMEM((2 * _D, _BPW), jnp.float32),
            pltpu.SemaphoreType.DMA,
        ],
    )
    def body(sid_hbm, cid_hbm, semb_hbm, cemb_hbm, out_hbm,
             sidx, cidx, sv, crows, combt, sem):
        wid = lax.axis_index("s") * _NC + lax.axis_index("c")
        base = wid * _BPW
        pltpu.sync_copy(cid_hbm.at[wid], cidx)
        copies = []
        for j in range(_NCH):
            copies.append(pltpu.async_copy(
                cemb_hbm.at[cidx.at[pl.ds(j * _CHUNK, _CHUNK)]],
                crows.at[pl.ds(j * _CHUNK, _CHUNK)], sem))
        pltpu.sync_copy(sid_hbm.at[wid], sidx)
        pltpu.sync_copy(semb_hbm, sv)

        cols = [jnp.full((16,), c, jnp.int32) for c in range(_D)]

        def sgroup(g, _):
            sid16 = sidx[pl.ds(g * 16, 16)]
            for d in range(_D):
                combt[d, pl.ds(g * 16, 16)] = plsc.load_gather(
                    sv, [sid16, cols[d]])
            return _

        lax.fori_loop(0, _NG, sgroup, None)
        for c in copies:
            c.wait()

        def cgroup(g, _):
            rows16 = lax.iota(jnp.int32, 16) + g * 16
            for d in range(_D):
                combt[_D + d, pl.ds(g * 16, 16)] = plsc.load_gather(
                    crows, [rows16, cols[d]])
            return _

        lax.fori_loop(0, _NG, cgroup, None)
        pltpu.sync_copy(combt, out_hbm.at[:, pl.ds(base, _BPW)])

    return body(sid, cid, semb, cemb)


def kernel(strategy_id, cause_index, strategy_emb, cause_emb):
    sid = strategy_id.astype(jnp.int32).reshape(_NW, _BPW)
    cid = cause_index.astype(jnp.int32).reshape(_NW, _BPW)
    return _embed(sid, cid, strategy_emb, cause_emb).T


# speedup vs baseline: 1.2724x; 1.2356x over previous
import functools

import jax
import jax.numpy as jnp
from jax import lax
from jax.experimental import pallas as pl
from jax.experimental.pallas import tpu as pltpu
from jax.experimental.pallas import tpu_sc as plsc

_B = 16384
_D = 32
_NC = 2
_NS = 16
_NW = _NC * _NS
_BPW = _B // _NW
_CHUNK = 128
_NCH = _BPW // _CHUNK
_NG = _BPW // 16
_DCH = 8
_NSL = _D // _DCH


def _embed(sid, cid, semb, cembt):
    mesh = plsc.VectorSubcoreMesh(core_axis_name="c", subcore_axis_name="s")

    @functools.partial(
        pl.kernel,
        mesh=mesh,
        out_type=jax.ShapeDtypeStruct((2 * _D, _B), jnp.float32),
        compiler_params=pltpu.CompilerParams(
            use_tc_tiling_on_sc=False, needs_layout_passes=False),
        scratch_types=[
            pltpu.VMEM((_BPW,), jnp.int32),
            pltpu.VMEM((_BPW,), jnp.int32),
            pltpu.VMEM((8, _D), jnp.float32),
            pltpu.VMEM((2 * _D, _BPW), jnp.float32),
            pltpu.VMEM_SHARED((_DCH, 100000), jnp.float32),
            pltpu.SemaphoreType.DMA,
            pltpu.SemaphoreType.DMA,
        ],
    )
    def body(sid_hbm, cid_hbm, semb_hbm, cembt_hbm, out_hbm,
             sidx, cidx, sv, combt, slab, gsem, ssem):
        sub = lax.axis_index("s")
        wid = sub * _NC + lax.axis_index("c")
        base = wid * _BPW
        pltpu.sync_copy(cid_hbm.at[wid], cidx)
        pltpu.sync_copy(sid_hbm.at[wid], sidx)
        pltpu.sync_copy(semb_hbm, sv)

        cols = [jnp.full((16,), c, jnp.int32) for c in range(_D)]

        def sgroup(g, _):
            sid16 = sidx[pl.ds(g * 16, 16)]
            for d in range(_D):
                combt[d, pl.ds(g * 16, 16)] = plsc.load_gather(
                    sv, [sid16, cols[d]])
            return _

        lax.fori_loop(0, _NG, sgroup, None)

        for k in range(_NSL):
            @pl.when(sub < _DCH)
            def _():
                pltpu.async_copy(
                    cembt_hbm.at[k * _DCH + sub], slab.at[sub], ssem).wait()

            plsc.subcore_barrier()
            copies = []
            for d in range(_DCH):
                for j in range(_NCH):
                    copies.append(pltpu.async_copy(
                        slab.at[d].at[cidx.at[pl.ds(j * _CHUNK, _CHUNK)]],
                        combt.at[_D + k * _DCH + d,
                                 pl.ds(j * _CHUNK, _CHUNK)],
                        gsem))
            for c in copies:
                c.wait()
            plsc.subcore_barrier()

        pltpu.sync_copy(combt, out_hbm.at[:, pl.ds(base, _BPW)])

    return body(sid, cid, semb, cembt)


def kernel(strategy_id, cause_index, strategy_emb, cause_emb):
    sid = strategy_id.astype(jnp.int32).reshape(_NW, _BPW)
    cid = cause_index.astype(jnp.int32).reshape(_NW, _BPW)
    return _embed(sid, cid, strategy_emb, cause_emb.T).T


# speedup vs baseline: 1.3244x; 1.0409x over previous
import functools

import jax
import jax.numpy as jnp
from jax import lax
from jax.experimental import pallas as pl
from jax.experimental.pallas import tpu as pltpu
from jax.experimental.pallas import tpu_sc as plsc

_B = 16384
_D = 32
_NC = 2
_NS = 16
_NW = _NC * _NS
_BPW = _B // _NW
_CHUNK = 128
_NCH = _BPW // _CHUNK
_NG = _BPW // 16
_DCH = 4
_NSL = _D // _DCH


def _embed(sid, cid, semb, cembt):
    mesh = plsc.VectorSubcoreMesh(core_axis_name="c", subcore_axis_name="s")

    @functools.partial(
        pl.kernel,
        mesh=mesh,
        out_type=jax.ShapeDtypeStruct((2 * _D, _B), jnp.float32),
        compiler_params=pltpu.CompilerParams(
            use_tc_tiling_on_sc=False, needs_layout_passes=False),
        scratch_types=[
            pltpu.VMEM((_BPW,), jnp.int32),
            pltpu.VMEM((_BPW,), jnp.int32),
            pltpu.VMEM((8, _D), jnp.float32),
            pltpu.VMEM((2 * _D, _BPW), jnp.float32),
            pltpu.VMEM_SHARED((_DCH, 100000), jnp.float32),
            pltpu.VMEM_SHARED((_DCH, 100000), jnp.float32),
            pltpu.SemaphoreType.DMA,
            pltpu.SemaphoreType.DMA,
        ],
    )
    def body(sid_hbm, cid_hbm, semb_hbm, cembt_hbm, out_hbm,
             sidx, cidx, sv, combt, slab0, slab1, gsem, ssem):
        sub = lax.axis_index("s")
        wid = sub * _NC + lax.axis_index("c")
        base = wid * _BPW
        pltpu.sync_copy(cid_hbm.at[wid], cidx)
        pltpu.sync_copy(sid_hbm.at[wid], sidx)
        pltpu.sync_copy(semb_hbm, sv)

        cols = [jnp.full((16,), c, jnp.int32) for c in range(_D)]

        def sgroup(g, _):
            sid16 = sidx[pl.ds(g * 16, 16)]
            for d in range(_D):
                combt[d, pl.ds(g * 16, 16)] = plsc.load_gather(
                    sv, [sid16, cols[d]])
            return _

        lax.fori_loop(0, _NG, sgroup, None)

        slabs = [slab0, slab1]

        @pl.when(sub < _DCH)
        def _():
            pltpu.async_copy(cembt_hbm.at[sub], slab0.at[sub], ssem)

        for k in range(_NSL):
            cur = slabs[k % 2]

            @pl.when(sub < _DCH)
            def _():
                pltpu.make_async_copy(
                    cembt_hbm.at[k * _DCH + sub], cur.at[sub], ssem).wait()

            plsc.subcore_barrier()

            if k + 1 < _NSL:
                nxt = slabs[(k + 1) % 2]

                @pl.when(sub < _DCH)
                def _():
                    pltpu.async_copy(
                        cembt_hbm.at[(k + 1) * _DCH + sub],
                        nxt.at[sub], ssem)

            copies = []
            for d in range(_DCH):
                for j in range(_NCH):
                    copies.append(pltpu.async_copy(
                        cur.at[d].at[cidx.at[pl.ds(j * _CHUNK, _CHUNK)]],
                        combt.at[_D + k * _DCH + d,
                                 pl.ds(j * _CHUNK, _CHUNK)],
                        gsem))
            for c in copies:
                c.wait()
            plsc.subcore_barrier()

        pltpu.sync_copy(combt, out_hbm.at[:, pl.ds(base, _BPW)])

    return body(sid, cid, semb, cembt)


def kernel(strategy_id, cause_index, strategy_emb, cause_emb):
    sid = strategy_id.astype(jnp.int32).reshape(_NW, _BPW)
    cid = cause_index.astype(jnp.int32).reshape(_NW, _BPW)
    return _embed(sid, cid, strategy_emb, cause_emb.T).T
